# baseline (speedup 1.0000x reference)
import functools

import jax
import jax.numpy as jnp
from jax.experimental import pallas as pl
from jax.experimental.pallas import tpu as pltpu

T, D, H, O, E = 2048, 768, 2048, 768, 8
TT = 256
EPAD = 128
NT = T // TT


def _moe_body(x_ref, wg_ref, bg_ref, w1_ref, b1_ref, w2_ref, b2_ref,
              out_ref):
    t = pl.program_id(0)
    e = pl.program_id(1)

    xt = x_ref[...]
    logits = jnp.dot(xt, wg_ref[...], preferred_element_type=jnp.float32)
    logits = logits + bg_ref[0:1, :]
    lane = jax.lax.broadcasted_iota(jnp.int32, (TT, EPAD), 1)
    neg = jnp.float32(-1e30)
    logits = jnp.where(lane < E, logits, neg)
    m = jnp.max(logits, axis=1, keepdims=True)
    p = jnp.exp(logits - m)
    p = jnp.where(lane < E, p, 0.0)
    s = jnp.sum(p, axis=1, keepdims=True)
    probs = p / s

    big = jnp.int32(EPAD + 1)
    v1 = jnp.max(probs, axis=1, keepdims=True)
    i1 = jnp.min(jnp.where(probs == v1, lane, big), axis=1, keepdims=True)
    masked = jnp.where(lane == i1, neg, probs)
    v2 = jnp.max(masked, axis=1, keepdims=True)
    i2 = jnp.min(jnp.where(masked == v2, lane, big), axis=1, keepdims=True)
    denom = jnp.maximum(v1 + v2, 1e-12)
    g = (jnp.where(i1 == e, v1, 0.0) + jnp.where(i2 == e, v2, 0.0)) / denom

    h = jnp.dot(xt, w1_ref[0], preferred_element_type=jnp.float32)
    h = jnp.maximum(h + b1_ref[0], 0.0)
    y = jnp.dot(h, w2_ref[0], preferred_element_type=jnp.float32)
    y = y + b2_ref[0]
    contrib = g * y

    @pl.when(e == 0)
    def _init():
        out_ref[...] = contrib

    @pl.when(e > 0)
    def _acc():
        out_ref[...] += contrib


@functools.partial(jax.jit, static_argnums=())
def _moe_fused(x2, wgp, bgp, W1, b1r, W2, b2r):
    return pl.pallas_call(
        _moe_body,
        grid=(NT, E),
        in_specs=[
            pl.BlockSpec((TT, D), lambda t, e: (t, 0)),
            pl.BlockSpec((D, EPAD), lambda t, e: (0, 0)),
            pl.BlockSpec((8, EPAD), lambda t, e: (0, 0)),
            pl.BlockSpec((1, D, H), lambda t, e: (e, 0, 0)),
            pl.BlockSpec((1, 1, H), lambda t, e: (e, 0, 0)),
            pl.BlockSpec((1, H, O), lambda t, e: (e, 0, 0)),
            pl.BlockSpec((1, 1, O), lambda t, e: (e, 0, 0)),
        ],
        out_specs=pl.BlockSpec((TT, O), lambda t, e: (t, 0)),
        out_shape=jax.ShapeDtypeStruct((T, O), jnp.float32),
        compiler_params=pltpu.CompilerParams(
            dimension_semantics=("arbitrary", "arbitrary"),
        ),
    )(x2, wgp, bgp, W1, b1r, W2, b2r)


def kernel(x, Wg, bg, W1, b1, W2, b2, num_experts_per_tok):
    x2 = x.reshape(T, D)
    wgp = jnp.pad(Wg, ((0, 0), (0, EPAD - E)))
    bgp = jnp.tile(jnp.pad(bg, (0, EPAD - E))[None, :], (8, 1))
    b1r = b1.reshape(E, 1, H)
    b2r = b2.reshape(E, 1, O)
    out = _moe_fused(x2, wgp, bgp, W1, b1r, W2, b2r)
    return out.reshape(1, T, O)


# speedup vs baseline: 1.1380x; 1.1380x over previous
import functools

import jax
import jax.numpy as jnp
from jax.experimental import pallas as pl
from jax.experimental.pallas import tpu as pltpu

T, D, H, O, E = 2048, 768, 2048, 768, 8
TT = 256
EPAD = 128
NT = T // TT


def _moe_body(x_ref, wg_ref, bg_ref, w1_ref, b1_ref, w2_ref, b2_ref,
              out_ref):
    t = pl.program_id(0)
    e = pl.program_id(1)

    xt = x_ref[...]
    logits = jnp.dot(xt, wg_ref[...], preferred_element_type=jnp.float32)
    logits = logits + bg_ref[0:1, :]
    lane = jax.lax.broadcasted_iota(jnp.int32, (TT, EPAD), 1)
    neg = jnp.float32(-1e30)
    logits = jnp.where(lane < E, logits, neg)
    m = jnp.max(logits, axis=1, keepdims=True)
    p = jnp.exp(logits - m)
    p = jnp.where(lane < E, p, 0.0)
    s = jnp.sum(p, axis=1, keepdims=True)
    probs = p / s

    big = jnp.int32(EPAD + 1)
    v1 = jnp.max(probs, axis=1, keepdims=True)
    i1 = jnp.min(jnp.where(probs == v1, lane, big), axis=1, keepdims=True)
    masked = jnp.where(lane == i1, neg, probs)
    v2 = jnp.max(masked, axis=1, keepdims=True)
    i2 = jnp.min(jnp.where(masked == v2, lane, big), axis=1, keepdims=True)
    denom = jnp.maximum(v1 + v2, 1e-12)
    g = (jnp.where(i1 == e, v1, 0.0) + jnp.where(i2 == e, v2, 0.0)) / denom

    h = jnp.dot(xt.astype(jnp.bfloat16), w1_ref[0],
                preferred_element_type=jnp.float32)
    h = jnp.maximum(h + b1_ref[0], 0.0)
    y = jnp.dot(h.astype(jnp.bfloat16), w2_ref[0],
                preferred_element_type=jnp.float32)
    y = y + b2_ref[0]
    contrib = g * y

    @pl.when(e == 0)
    def _init():
        out_ref[...] = contrib

    @pl.when(e > 0)
    def _acc():
        out_ref[...] += contrib


@functools.partial(jax.jit, static_argnums=())
def _moe_fused(x2, wgp, bgp, W1, b1r, W2, b2r):
    return pl.pallas_call(
        _moe_body,
        grid=(NT, E),
        in_specs=[
            pl.BlockSpec((TT, D), lambda t, e: (t, 0)),
            pl.BlockSpec((D, EPAD), lambda t, e: (0, 0)),
            pl.BlockSpec((8, EPAD), lambda t, e: (0, 0)),
            pl.BlockSpec((1, D, H), lambda t, e: (e, 0, 0)),
            pl.BlockSpec((1, 1, H), lambda t, e: (e, 0, 0)),
            pl.BlockSpec((1, H, O), lambda t, e: (e, 0, 0)),
            pl.BlockSpec((1, 1, O), lambda t, e: (e, 0, 0)),
        ],
        out_specs=pl.BlockSpec((TT, O), lambda t, e: (t, 0)),
        out_shape=jax.ShapeDtypeStruct((T, O), jnp.float32),
        compiler_params=pltpu.CompilerParams(
            dimension_semantics=("arbitrary", "arbitrary"),
        ),
    )(x2, wgp, bgp, W1, b1r, W2, b2r)


def kernel(x, Wg, bg, W1, b1, W2, b2, num_experts_per_tok):
    x2 = x.reshape(T, D)
    wgp = jnp.pad(Wg, ((0, 0), (0, EPAD - E)))
    bgp = jnp.tile(jnp.pad(bg, (0, EPAD - E))[None, :], (8, 1))
    b1r = b1.reshape(E, 1, H)
    b2r = b2.reshape(E, 1, O)
    out = _moe_fused(x2, wgp, bgp, W1.astype(jnp.bfloat16), b1r,
                     W2.astype(jnp.bfloat16), b2r)
    return out.reshape(1, T, O)


# speedup vs baseline: 1.3996x; 1.2299x over previous
import functools

import jax
import jax.numpy as jnp
from jax import lax
from jax.experimental import pallas as pl
from jax.experimental.pallas import tpu as pltpu
from jax.experimental.pallas import tpu_sc as plsc

T, D, H, O, E = 2048, 768, 2048, 768, 8
EPAD = 128
RT = 256
TT2 = 256
NP = T * 2
P = NP + E * (TT2 - 1) // TT2 * TT2 + (TT2 - NP % TT2) % TT2
P = 6144
NTILE = P // TT2
NW = 32
APW = NP // NW
NCH = NP // 16


def _router_body(wgt_ref, bg_ref, x_ref, out_ref):
    lg = lax.dot_general(wgt_ref[...], x_ref[...],
                         (((1,), (1,)), ((), ())),
                         preferred_element_type=jnp.float32)
    lg = lg + bg_ref[:, 0:1]
    row = lax.broadcasted_iota(jnp.int32, (EPAD, RT), 0)
    neg = jnp.float32(-1e30)
    lg = jnp.where(row < E, lg, neg)
    m = jnp.max(lg, axis=0, keepdims=True)
    p = jnp.exp(lg - m)
    p = jnp.where(row < E, p, 0.0)
    probs = p / jnp.sum(p, axis=0, keepdims=True)

    big = jnp.int32(EPAD + 1)
    v1 = jnp.max(probs, axis=0, keepdims=True)
    i1 = jnp.min(jnp.where(probs == v1, row, big), axis=0, keepdims=True)
    masked = jnp.where(row == i1, neg, probs)
    v2 = jnp.max(masked, axis=0, keepdims=True)
    i2 = jnp.min(jnp.where(masked == v2, row, big), axis=0, keepdims=True)
    denom = jnp.maximum(v1 + v2, 1e-12)
    out_ref[...] = jnp.concatenate(
        [v1 / denom, v2 / denom,
         i1.astype(jnp.float32), i2.astype(jnp.float32),
         jnp.zeros((4, RT), jnp.float32)], axis=0)


def _router(x2, wgt, bgc):
    return pl.pallas_call(
        _router_body,
        grid=(T // RT,),
        in_specs=[
            pl.BlockSpec((EPAD, D), lambda t: (0, 0)),
            pl.BlockSpec((EPAD, 8), lambda t: (0, 0)),
            pl.BlockSpec((RT, D), lambda t: (t, 0)),
        ],
        out_specs=pl.BlockSpec((8, RT), lambda t: (0, t)),
        out_shape=jax.ShapeDtypeStruct((8, T), jnp.float32),
    )(wgt, bgc, x2)


def _lane16():
    return lax.iota(jnp.int32, 16)


def _c16(v):
    return jnp.full((16,), v, jnp.int32)


def _bcast(s):
    return lax.broadcast_in_dim(s, (16,), ())


def _gather16(v, idx):
    return lax.gather(
        v, idx[:, None],
        lax.GatherDimensionNumbers(offset_dims=(), collapsed_slice_dims=(0,),
                                   start_index_map=(0,)),
        (1,), mode=lax.GatherScatterMode.PROMISE_IN_BOUNDS)


def _ks_cumsum(x, lanes, zero16):
    for k in (1, 2, 4, 8):
        sh = _gather16(x, jnp.maximum(lanes - _c16(k), zero16))
        x = x + jnp.where(lanes >= _c16(k), sh, zero16)
    return x


def _i01(x, zero16, one16):
    return jnp.minimum(jnp.maximum(x, zero16), one16)


def _sel8(v16, ids, zero16, one16):
    out = zero16
    for e in range(E):
        eq = _i01(one16 - jnp.abs(ids - _c16(e)), zero16, one16)
        out = out + eq * _bcast(v16[e])
    return out


def _sc_count_body(rf_hbm, cnt_hbm, rel_hbm, ids_v, rel_v, cw_v):
    wid = lax.axis_index("s") * 2 + lax.axis_index("c")
    pltpu.sync_copy(rf_hbm.at[pl.ds(2 * T + wid * APW, APW)], ids_v)

    lanes = _lane16()
    zero16 = jnp.zeros((16,), jnp.int32)

    run = zero16
    for k in range(8):
        ids = ids_v[pl.ds(k * 16, 16)].astype(jnp.int32)
        relk = zero16
        newrun = run
        for e in range(E):
            msk = ids == _c16(e)
            cs = _ks_cumsum(jnp.where(msk, jnp.ones((16,), jnp.int32),
                                      zero16), lanes, zero16)
            relk = jnp.where(msk, _bcast(run[e]) + cs - _c16(1), relk)
            newrun = newrun + jnp.where(lanes == _c16(e), _bcast(cs[15]),
                                        zero16)
        run = newrun
        rel_v[pl.ds(k * 16, 16)] = relk

    pltpu.sync_copy(rel_v, rel_hbm.at[pl.ds(wid * APW, APW)])
    cw_v[...] = run
    pltpu.sync_copy(cw_v, cnt_hbm.at[wid])


def _sc_count(rflat):
    mesh = plsc.VectorSubcoreMesh(core_axis_name="c", subcore_axis_name="s")
    f = pl.kernel(
        _sc_count_body,
        mesh=mesh,
        out_type=[
            jax.ShapeDtypeStruct((NW, 16), jnp.int32),
            jax.ShapeDtypeStruct((NP,), jnp.int32),
        ],
        scratch_types=[
            pltpu.VMEM((APW,), jnp.float32),
            pltpu.VMEM((APW,), jnp.int32),
            pltpu.VMEM((16,), jnp.int32),
        ],
    )
    return f(rflat)


def _sc_scatter_body(rf_hbm, x_hbm, cnt_hbm, rel_hbm,
                     xs_hbm, pos_hbm, wrow_hbm, et_hbm, ea_hbm,
                     cnt_v, ids_v, rel_v, myw_v, pos_v, xslab_v, meta_v, sem):
    wid = lax.axis_index("s") * 2 + lax.axis_index("c")
    jrow = wid // 16
    tbase = (wid % 16) * 128

    pltpu.sync_copy(cnt_hbm, cnt_v)
    pltpu.sync_copy(rf_hbm.at[pl.ds(2 * T + wid * APW, APW)], ids_v)
    pltpu.sync_copy(rel_hbm.at[pl.ds(wid * APW, APW)], rel_v)
    pltpu.sync_copy(rf_hbm.at[pl.ds(jrow * T + tbase, APW)], myw_v)

    lanes = _lane16()
    zero16 = jnp.zeros((16,), jnp.int32)
    one16 = jnp.ones((16,), jnp.int32)

    widv = _bcast(wid)
    total = zero16
    prior = zero16
    for w in range(NW):
        cw = cnt_v[w]
        total = total + cw
        lt = _i01(widv - _c16(w), zero16, one16)
        prior = prior + lt * cw

    is_e = _i01(_c16(E) - lanes, zero16, one16)
    rc = is_e * ((total + _c16(TT2 - 1)) & _c16(-TT2))
    offs_excl = zero16
    for e in range(E - 1):
        gt = _i01(lanes - _c16(e), zero16, one16)
        offs_excl = offs_excl + gt * _bcast(rc[e])
    base = offs_excl + prior

    for k in range(8):
        ids = ids_v[pl.ds(k * 16, 16)].astype(jnp.int32)
        pos_v[pl.ds(k * 16, 16)] = rel_v[pl.ds(k * 16, 16)] + \
            _sel8(base, ids, zero16, one16)

    pltpu.sync_copy(pos_v, pos_hbm.at[pl.ds(wid * APW, APW)])

    pltpu.sync_copy(x_hbm.at[pl.ds(tbase, 128)], xslab_v)
    pltpu.async_copy(xslab_v, xs_hbm.at[pos_v], sem).wait()
    pltpu.async_copy(myw_v, wrow_hbm.at[pos_v], sem).wait()

    @pl.when(wid == 0)
    def _meta():
        offs_incl = offs_excl + rc
        reach = offs_excl + total
        for half in range(2):
            toff = (lanes + _c16(half * 16)) * _c16(TT2)
            cnt = zero16
            for e in range(E):
                ge = _i01(toff - _bcast(offs_incl[e]) + one16,
                          zero16, one16)
                cnt = cnt + ge
            et = jnp.minimum(cnt, _c16(E - 1))
            bound = _sel8(reach, et, zero16, one16)
            meta_v[pl.ds(half * 16, 16)] = et
            meta_v[pl.ds(32 + half * 16, 16)] = _i01(bound - toff,
                                                     zero16, one16)
        pltpu.sync_copy(meta_v.at[pl.ds(0, 32)], et_hbm)
        pltpu.sync_copy(meta_v.at[pl.ds(32, 32)], ea_hbm)


def _sc_scatter(rflat, x2, cnts, rel):
    mesh = plsc.VectorSubcoreMesh(core_axis_name="c", subcore_axis_name="s")
    f = pl.kernel(
        _sc_scatter_body,
        mesh=mesh,
        out_type=[
            jax.ShapeDtypeStruct((P, D), jnp.float32),
            jax.ShapeDtypeStruct((NP,), jnp.int32),
            jax.ShapeDtypeStruct((P,), jnp.float32),
            jax.ShapeDtypeStruct((32,), jnp.int32),
            jax.ShapeDtypeStruct((32,), jnp.int32),
        ],
        scratch_types=[
            pltpu.VMEM((NW, 16), jnp.int32),
            pltpu.VMEM((APW,), jnp.float32),
            pltpu.VMEM((APW,), jnp.int32),
            pltpu.VMEM((APW,), jnp.float32),
            pltpu.VMEM((APW,), jnp.int32),
            pltpu.VMEM((128, D), jnp.float32),
            pltpu.VMEM((64,), jnp.int32),
            pltpu.SemaphoreType.DMA,
        ],
    )
    return f(rflat, x2, cnts, rel)


def _ffn_body(et_ref, ea_ref, xs_ref, w1_ref, b1_ref, w2_ref, b2_ref,
              wr_ref, ys_ref):
    i = pl.program_id(0)

    @pl.when(ea_ref[i] != 0)
    def _go():
        xt = xs_ref[...].astype(jnp.bfloat16)
        h = jnp.dot(xt, w1_ref[0], preferred_element_type=jnp.float32)
        h = jnp.maximum(h + b1_ref[0], 0.0)
        y = jnp.dot(h.astype(jnp.bfloat16), w2_ref[0],
                    preferred_element_type=jnp.float32)
        y = y + b2_ref[0]
        wcol = jnp.transpose(wr_ref[0], (1, 0))
        ys_ref[...] = y * wcol


def _ffn(et, ea, xs, W1b, b1r, W2b, b2r, wrow3):
    grid_spec = pltpu.PrefetchScalarGridSpec(
        num_scalar_prefetch=2,
        grid=(NTILE,),
        in_specs=[
            pl.BlockSpec((TT2, D), lambda i, et, ea: (i, 0)),
            pl.BlockSpec((1, D, H), lambda i, et, ea: (et[i], 0, 0)),
            pl.BlockSpec((1, 1, H), lambda i, et, ea: (et[i], 0, 0)),
            pl.BlockSpec((1, H, O), lambda i, et, ea: (et[i], 0, 0)),
            pl.BlockSpec((1, 1, O), lambda i, et, ea: (et[i], 0, 0)),
            pl.BlockSpec((1, 1, TT2), lambda i, et, ea: (i, 0, 0)),
        ],
        out_specs=pl.BlockSpec((TT2, O), lambda i, et, ea: (i, 0)),
    )
    return pl.pallas_call(
        _ffn_body,
        grid_spec=grid_spec,
        out_shape=jax.ShapeDtypeStruct((P, O), jnp.float32),
        compiler_params=pltpu.CompilerParams(
            dimension_semantics=("arbitrary",),
        ),
    )(et, ea, xs, W1b, b1r, W2b, b2r, wrow3)


def _sc_combine_body(ys_hbm, pos_hbm, out_hbm, p0_v, p1_v, buf0_v, buf1_v, sem):
    wid = lax.axis_index("s") * 2 + lax.axis_index("c")
    base = wid * 64
    pltpu.sync_copy(pos_hbm.at[pl.ds(base, 64)], p0_v)
    pltpu.sync_copy(pos_hbm.at[pl.ds(T + base, 64)], p1_v)
    pltpu.async_copy(ys_hbm.at[p0_v], buf0_v, sem).wait()
    pltpu.async_copy(ys_hbm.at[p1_v], buf1_v, sem).wait()

    def row_add(r, _):
        for k in range(O // 16):
            buf0_v[r, pl.ds(k * 16, 16)] = (
                buf0_v[r, pl.ds(k * 16, 16)] + buf1_v[r, pl.ds(k * 16, 16)])
        return 0

    lax.fori_loop(0, 64, row_add, 0)
    pltpu.sync_copy(buf0_v, out_hbm.at[pl.ds(base, 64)])


def _sc_combine(ys, pos):
    mesh = plsc.VectorSubcoreMesh(core_axis_name="c", subcore_axis_name="s")
    f = pl.kernel(
        _sc_combine_body,
        mesh=mesh,
        out_type=jax.ShapeDtypeStruct((T, O), jnp.float32),
        scratch_types=[
            pltpu.VMEM((64,), jnp.int32),
            pltpu.VMEM((64,), jnp.int32),
            pltpu.VMEM((64, O), jnp.float32),
            pltpu.VMEM((64, O), jnp.float32),
            pltpu.SemaphoreType.DMA,
        ],
    )
    return f(ys, pos)


@jax.jit
def _moe(x2, wgt, bgc, W1b, b1r, W2b, b2r):
    route = _router(x2, wgt, bgc)
    rflat = route.reshape(8 * T)
    cnts, rel = _sc_count(rflat)
    xs, pos, wrow, et, ea = _sc_scatter(rflat, x2, cnts, rel)
    ys = _ffn(et, ea, xs, W1b, b1r, W2b, b2r, wrow.reshape(NTILE, 1, TT2))
    return _sc_combine(ys, pos)


def kernel(x, Wg, bg, W1, b1, W2, b2, num_experts_per_tok):
    x2 = x.reshape(T, D)
    wgt = jnp.pad(Wg.T, ((0, EPAD - E), (0, 0)))
    bgc = jnp.tile(jnp.pad(bg, (0, EPAD - E))[:, None], (1, 8))
    out = _moe(x2, wgt, bgc,
               W1.astype(jnp.bfloat16), b1.reshape(E, 1, H),
               W2.astype(jnp.bfloat16), b2.reshape(E, 1, O))
    return out.reshape(1, T, O)
